# baseline (speedup 1.0000x reference)
import functools

import jax
import jax.numpy as jnp
from jax import lax
from jax.experimental import pallas as pl
from jax.experimental.pallas import tpu as pltpu
from jax.experimental.pallas import tpu_sc as plsc

EMBED_DIM = 32
PAD_DIM = 128
NUM_CORES = 2
NUM_SUBCORES = 16
NUM_WORKERS = NUM_CORES * NUM_SUBCORES
GRP = 128
K = 2
CHUNK = K * GRP

_MESH = plsc.VectorSubcoreMesh(
    core_axis_name="c", subcore_axis_name="s",
    num_cores=NUM_CORES, num_subcores=NUM_SUBCORES,
)


def _make_gather(nchunks: int):
  @functools.partial(
      pl.kernel,
      mesh=_MESH,
      compiler_params=pltpu.CompilerParams(use_tc_tiling_on_sc=False),
      out_type=jax.ShapeDtypeStruct(
          (NUM_WORKERS, nchunks, K, GRP, EMBED_DIM), jnp.float32),
      scratch_types=[
          pltpu.VMEM((nchunks * K, GRP), jnp.int32),
          pltpu.VMEM((2, K, GRP, PAD_DIM), jnp.float32),
          pltpu.SemaphoreType.DMA,
          pltpu.SemaphoreType.DMA,
      ],
  )
  def gather_kernel(idx_hbm, table_hbm, out_hbm, idx_v, rows_v, gsem, osem):
    wid = lax.axis_index("s") * NUM_CORES + lax.axis_index("c")

    def fire(g, slot):
      for j in range(K):
        pltpu.async_copy(table_hbm.at[idx_v.at[g * K + j]],
                         rows_v.at[slot, j], gsem)

    def drain_gathers(slot):
      for j in range(K):
        pltpu.make_async_copy(table_hbm.at[idx_v.at[j]],
                              rows_v.at[slot, j], gsem).wait()

    def drain_write():
      pltpu.make_async_copy(rows_v.at[0, :, :, pl.ds(0, EMBED_DIM)],
                            out_hbm.at[wid, 0], osem).wait()

    pltpu.sync_copy(idx_hbm.at[wid], idx_v)
    fire(0, 0)

    @pl.loop(0, nchunks)
    def _chunk(g):
      s = g % 2
      has_next = g + 1 < nchunks

      @pl.when(jnp.logical_and(g >= 1, has_next))
      def _():
        drain_write()

      @pl.when(has_next)
      def _():
        fire(g + 1, 1 - s)

      drain_gathers(s)
      pltpu.async_copy(rows_v.at[s, :, :, pl.ds(0, EMBED_DIM)],
                       out_hbm.at[wid, g], osem)

    drain_write()
    drain_write()

  return gather_kernel


def kernel(skills, embedding_table):
  batch, hist = skills.shape
  total = batch * hist
  assert total % (NUM_WORKERS * CHUNK) == 0
  nchunks = total // (NUM_WORKERS * CHUNK)
  idx = skills.reshape(NUM_WORKERS, nchunks * K, GRP)
  vocab = embedding_table.shape[0]
  pad_rows = (-vocab) % 8
  table_p = jnp.pad(embedding_table,
                    ((0, pad_rows), (0, PAD_DIM - EMBED_DIM)))
  out = _make_gather(nchunks)(idx, table_p)
  return out.reshape(batch, hist, EMBED_DIM)


# speedup vs baseline: 1.0864x; 1.0864x over previous
import functools

import jax
import jax.numpy as jnp
from jax import lax
from jax.experimental import pallas as pl
from jax.experimental.pallas import tpu as pltpu
from jax.experimental.pallas import tpu_sc as plsc

EMBED_DIM = 32
NUM_CORES = 2
NUM_SUBCORES = 16
NUM_WORKERS = NUM_CORES * NUM_SUBCORES
BATCH = 16384
HIST = 50
HIST_PAD = 128
VPAD = 100096
BC = 64
NCHUNK = BATCH // BC

_MESH = plsc.VectorSubcoreMesh(
    core_axis_name="c", subcore_axis_name="s",
    num_cores=NUM_CORES, num_subcores=NUM_SUBCORES,
)


@functools.partial(
    pl.kernel,
    mesh=_MESH,
    compiler_params=pltpu.CompilerParams(use_tc_tiling_on_sc=False, needs_layout_passes=False),
    out_type=jax.ShapeDtypeStruct((HIST, EMBED_DIM, BATCH), jnp.float32),
    scratch_types=[
        pltpu.VMEM((VPAD,), jnp.float32),
        pltpu.VMEM((2, BC, HIST_PAD), jnp.int32),
        pltpu.VMEM((2, HIST, BC), jnp.float32),
        pltpu.SemaphoreType.DMA,
        pltpu.SemaphoreType.DMA,
        pltpu.SemaphoreType.DMA,
    ],
)
def _gather_t(idx_hbm, table_hbm, out_hbm, row_v, idx_v, out_v,
              rsem, isem, osem):
  d = lax.axis_index("s") * NUM_CORES + lax.axis_index("c")

  def fire_idx(g, slot):
    pltpu.async_copy(idx_hbm.at[pl.ds(g * BC, BC), :], idx_v.at[slot], isem)

  def wait_idx(slot):
    pltpu.make_async_copy(idx_hbm.at[pl.ds(0, BC), :], idx_v.at[slot],
                          isem).wait()

  def wait_out():
    pltpu.make_async_copy(out_v.at[0], out_hbm.at[:, 0, pl.ds(0, BC)],
                          osem).wait()

  pltpu.async_copy(table_hbm.at[d], row_v, rsem)
  fire_idx(0, 0)
  pltpu.make_async_copy(table_hbm.at[d], row_v, rsem).wait()

  rows = [lax.iota(jnp.int32, 16) + bs * 16 for bs in range(BC // 16)]

  @pl.loop(0, NCHUNK)
  def _chunk(g):
    s = g % 2

    @pl.when(g + 1 < NCHUNK)
    def _():
      fire_idx(g + 1, 1 - s)

    wait_idx(s)

    @pl.when(g >= 2)
    def _():
      wait_out()

    @pl.loop(0, HIST)
    def _h(h):
      cols = jnp.full((16,), h, jnp.int32)
      for bs in range(BC // 16):
        v = plsc.load_gather(idx_v.at[s], [rows[bs], cols])
        vals = plsc.load_gather(row_v, [v])
        out_v[s, h, pl.ds(bs * 16, 16)] = vals

    pltpu.async_copy(out_v.at[s], out_hbm.at[:, d, pl.ds(g * BC, BC)], osem)

  wait_out()
  wait_out()


def kernel(skills, embedding_table):
  idx_p = jnp.pad(skills, ((0, 0), (0, HIST_PAD - HIST)))
  table_t = jnp.pad(embedding_table.T,
                    ((0, 0), (0, VPAD - embedding_table.shape[0])))
  out = _gather_t(idx_p, table_t)
  return jnp.transpose(out, (2, 0, 1))


# speedup vs baseline: 2.0807x; 1.9153x over previous
import functools

import jax
import jax.numpy as jnp
from jax import lax
from jax.experimental import pallas as pl
from jax.experimental.pallas import tpu as pltpu
from jax.experimental.pallas import tpu_sc as plsc

EMBED_DIM = 32
NUM_CORES = 2
NUM_SUBCORES = 16
NUM_WORKERS = NUM_CORES * NUM_SUBCORES
BATCH = 16384
HIST = 50
HIST_PAD = 128
VPAD = 100096
BC = 64
NCHUNK = BATCH // BC

_MESH = plsc.VectorSubcoreMesh(
    core_axis_name="c", subcore_axis_name="s",
    num_cores=NUM_CORES, num_subcores=NUM_SUBCORES,
)


@functools.partial(
    pl.kernel,
    mesh=_MESH,
    compiler_params=pltpu.CompilerParams(use_tc_tiling_on_sc=False, needs_layout_passes=False),
    out_type=jax.ShapeDtypeStruct((HIST, EMBED_DIM, BATCH), jnp.float32),
    scratch_types=[
        pltpu.VMEM((VPAD,), jnp.float32),
        pltpu.VMEM((2, BC, HIST_PAD), jnp.int32),
        pltpu.VMEM((2, HIST, BC), jnp.float32),
        pltpu.SemaphoreType.DMA,
        pltpu.SemaphoreType.DMA,
        pltpu.SemaphoreType.DMA,
    ],
)
def _gather_t(idx_hbm, table_hbm, out_hbm, row_v, idx_v, out_v,
              rsem, isem, osem):
  d = lax.axis_index("s") * NUM_CORES + lax.axis_index("c")

  def fire_idx(g, slot):
    pltpu.async_copy(idx_hbm.at[pl.ds(g * BC, BC), :], idx_v.at[slot], isem)

  def wait_idx(slot):
    pltpu.make_async_copy(idx_hbm.at[pl.ds(0, BC), :], idx_v.at[slot],
                          isem).wait()

  def wait_out():
    pltpu.make_async_copy(out_v.at[0], out_hbm.at[:, 0, pl.ds(0, BC)],
                          osem).wait()

  pltpu.async_copy(table_hbm.at[d], row_v, rsem)
  fire_idx(0, 0)
  pltpu.make_async_copy(table_hbm.at[d], row_v, rsem).wait()

  rows = [lax.iota(jnp.int32, 16) + bs * 16 for bs in range(BC // 16)]

  @pl.loop(0, NCHUNK)
  def _chunk(g):
    s = g % 2

    @pl.when(g + 1 < NCHUNK)
    def _():
      fire_idx(g + 1, 1 - s)

    wait_idx(s)

    @pl.when(g >= 2)
    def _():
      wait_out()

    idx_s = idx_v.at[s]
    out_s = out_v.at[s]

    @plsc.parallel_loop(0, HIST, unroll=2)
    def _h(h):
      cols = jnp.full((16,), h, jnp.int32)
      for bs in range(BC // 16):
        v = plsc.load_gather(idx_s, [rows[bs], cols])
        vals = plsc.load_gather(row_v, [v])
        out_s[h, pl.ds(bs * 16, 16)] = vals

    pltpu.async_copy(out_v.at[s], out_hbm.at[:, d, pl.ds(g * BC, BC)], osem)

  wait_out()
  wait_out()


def kernel(skills, embedding_table):
  idx_p = jnp.pad(skills, ((0, 0), (0, HIST_PAD - HIST)))
  table_t = jnp.pad(embedding_table.T,
                    ((0, 0), (0, VPAD - embedding_table.shape[0])))
  out = _gather_t(idx_p, table_t)
  return jnp.transpose(out, (2, 0, 1))


# speedup vs baseline: 2.0933x; 1.0061x over previous
import functools

import jax
import jax.numpy as jnp
from jax import lax
from jax.experimental import pallas as pl
from jax.experimental.pallas import tpu as pltpu
from jax.experimental.pallas import tpu_sc as plsc

EMBED_DIM = 32
NUM_CORES = 2
NUM_SUBCORES = 16
NUM_WORKERS = NUM_CORES * NUM_SUBCORES
BATCH = 16384
HIST = 50
HIST_PAD = 128
VPAD = 100096
BC = 64
NCHUNK = BATCH // BC

_MESH = plsc.VectorSubcoreMesh(
    core_axis_name="c", subcore_axis_name="s",
    num_cores=NUM_CORES, num_subcores=NUM_SUBCORES,
)


@functools.partial(
    pl.kernel,
    mesh=_MESH,
    compiler_params=pltpu.CompilerParams(use_tc_tiling_on_sc=False, needs_layout_passes=False),
    out_type=jax.ShapeDtypeStruct((HIST, EMBED_DIM, BATCH), jnp.float32),
    scratch_types=[
        pltpu.VMEM((VPAD,), jnp.float32),
        pltpu.VMEM((2, BC, HIST_PAD), jnp.int32),
        pltpu.VMEM((2, HIST, BC), jnp.float32),
        pltpu.SemaphoreType.DMA,
        pltpu.SemaphoreType.DMA,
        pltpu.SemaphoreType.DMA,
    ],
)
def _gather_t(idx_hbm, table_hbm, out_hbm, row_v, idx_v, out_v,
              rsem, isem, osem):
  d = lax.axis_index("s") * NUM_CORES + lax.axis_index("c")

  def fire_idx(g, slot):
    pltpu.async_copy(idx_hbm.at[pl.ds(g * BC, BC), :], idx_v.at[slot], isem)

  def wait_idx(slot):
    pltpu.make_async_copy(idx_hbm.at[pl.ds(0, BC), :], idx_v.at[slot],
                          isem).wait()

  def wait_out():
    pltpu.make_async_copy(out_v.at[0], out_hbm.at[:, 0, pl.ds(0, BC)],
                          osem).wait()

  pltpu.async_copy(table_hbm.at[d], row_v, rsem)
  fire_idx(0, 0)
  pltpu.make_async_copy(table_hbm.at[d], row_v, rsem).wait()

  rows = [lax.iota(jnp.int32, 16) + bs * 16 for bs in range(BC // 16)]

  @pl.loop(0, NCHUNK)
  def _chunk(g):
    s = g % 2

    @pl.when(g + 1 < NCHUNK)
    def _():
      fire_idx(g + 1, 1 - s)

    wait_idx(s)

    @pl.when(g >= 2)
    def _():
      wait_out()

    idx_s = idx_v.at[s]
    out_s = out_v.at[s]

    @plsc.parallel_loop(0, HIST, unroll=5)
    def _h(h):
      cols = jnp.full((16,), h, jnp.int32)
      for bs in range(BC // 16):
        v = plsc.load_gather(idx_s, [rows[bs], cols])
        vals = plsc.load_gather(row_v, [v])
        out_s[h, pl.ds(bs * 16, 16)] = vals

    pltpu.async_copy(out_v.at[s], out_hbm.at[:, d, pl.ds(g * BC, BC)], osem)

  wait_out()
  wait_out()


def kernel(skills, embedding_table):
  idx_p = jnp.pad(skills, ((0, 0), (0, HIST_PAD - HIST)))
  table_t = jnp.pad(embedding_table.T,
                    ((0, 0), (0, VPAD - embedding_table.shape[0])))
  out = _gather_t(idx_p, table_t)
  return jnp.transpose(out, (2, 0, 1))


# speedup vs baseline: 3.7969x; 1.8138x over previous
import functools

import jax
import jax.numpy as jnp
from jax import lax
from jax.experimental import pallas as pl
from jax.experimental.pallas import tpu as pltpu
from jax.experimental.pallas import tpu_sc as plsc

EMBED_DIM = 32
NUM_CORES = 2
NUM_SUBCORES = 16
NUM_WORKERS = NUM_CORES * NUM_SUBCORES
BATCH = 16384
HIST = 50
HIST_PAD = 128
VPAD = 100096
BC = 128
NCHUNK = BATCH // BC

_MESH = plsc.VectorSubcoreMesh(
    core_axis_name="c", subcore_axis_name="s",
    num_cores=NUM_CORES, num_subcores=NUM_SUBCORES,
)


@functools.partial(
    pl.kernel,
    mesh=_MESH,
    compiler_params=pltpu.CompilerParams(use_tc_tiling_on_sc=False, needs_layout_passes=False),
    out_type=jax.ShapeDtypeStruct((HIST, EMBED_DIM, BATCH), jnp.float32),
    scratch_types=[
        pltpu.VMEM((VPAD,), jnp.float32),
        pltpu.VMEM((2, HIST, BC), jnp.int32),
        pltpu.VMEM((2, HIST, BC), jnp.float32),
        pltpu.SemaphoreType.DMA,
        pltpu.SemaphoreType.DMA,
        pltpu.SemaphoreType.DMA,
    ],
)
def _gather_t(idx_hbm, table_hbm, out_hbm, row_v, idx_v, out_v,
              rsem, isem, osem):
  d = lax.axis_index("s") * NUM_CORES + lax.axis_index("c")

  def fire_idx(g, slot):
    pltpu.async_copy(idx_hbm.at[:, pl.ds(g * BC, BC)], idx_v.at[slot], isem)

  def wait_idx(slot):
    pltpu.make_async_copy(idx_hbm.at[:, pl.ds(0, BC)], idx_v.at[slot],
                          isem).wait()

  def wait_out():
    pltpu.make_async_copy(out_v.at[0], out_hbm.at[:, 0, pl.ds(0, BC)],
                          osem).wait()

  pltpu.async_copy(table_hbm.at[d], row_v, rsem)
  fire_idx(0, 0)
  pltpu.make_async_copy(table_hbm.at[d], row_v, rsem).wait()

  @pl.loop(0, NCHUNK)
  def _chunk(g):
    s = g % 2

    @pl.when(g + 1 < NCHUNK)
    def _():
      fire_idx(g + 1, 1 - s)

    wait_idx(s)

    @pl.when(g >= 2)
    def _():
      wait_out()

    idx_s = idx_v.at[s]
    out_s = out_v.at[s]

    @plsc.parallel_loop(0, HIST, unroll=2)
    def _h(h):
      for bs in range(BC // 16):
        v = idx_s[h, pl.ds(bs * 16, 16)]
        vals = plsc.load_gather(row_v, [v])
        out_s[h, pl.ds(bs * 16, 16)] = vals

    pltpu.async_copy(out_v.at[s], out_hbm.at[:, d, pl.ds(g * BC, BC)], osem)

  wait_out()
  wait_out()


def kernel(skills, embedding_table):
  idx_p = skills.T
  table_t = jnp.pad(embedding_table.T,
                    ((0, 0), (0, VPAD - embedding_table.shape[0])))
  out = _gather_t(idx_p, table_t)
  return jnp.transpose(out, (2, 0, 1))


# speedup vs baseline: 3.8032x; 1.0016x over previous
import functools

import jax
import jax.numpy as jnp
from jax import lax
from jax.experimental import pallas as pl
from jax.experimental.pallas import tpu as pltpu
from jax.experimental.pallas import tpu_sc as plsc

EMBED_DIM = 32
NUM_CORES = 2
NUM_SUBCORES = 16
NUM_WORKERS = NUM_CORES * NUM_SUBCORES
BATCH = 16384
HIST = 50
HIST_PAD = 128
VPAD = 100096
BC = 128
NCHUNK = BATCH // BC

_MESH = plsc.VectorSubcoreMesh(
    core_axis_name="c", subcore_axis_name="s",
    num_cores=NUM_CORES, num_subcores=NUM_SUBCORES,
)


@functools.partial(
    pl.kernel,
    mesh=_MESH,
    compiler_params=pltpu.CompilerParams(use_tc_tiling_on_sc=False, needs_layout_passes=False),
    out_type=jax.ShapeDtypeStruct((HIST, EMBED_DIM, BATCH), jnp.float32),
    scratch_types=[
        pltpu.VMEM((VPAD,), jnp.float32),
        pltpu.VMEM((2, HIST, BC), jnp.int32),
        pltpu.VMEM((2, HIST, BC), jnp.float32),
        pltpu.SemaphoreType.DMA,
        pltpu.SemaphoreType.DMA,
        pltpu.SemaphoreType.DMA,
    ],
)
def _gather_t(idx_hbm, table_hbm, out_hbm, row_v, idx_v, out_v,
              rsem, isem, osem):
  d = lax.axis_index("s") * NUM_CORES + lax.axis_index("c")

  def fire_idx(g, slot):
    pltpu.async_copy(idx_hbm.at[:, pl.ds(g * BC, BC)], idx_v.at[slot], isem)

  def wait_idx(slot):
    pltpu.make_async_copy(idx_hbm.at[:, pl.ds(0, BC)], idx_v.at[slot],
                          isem).wait()

  def wait_out():
    pltpu.make_async_copy(out_v.at[0], out_hbm.at[:, 0, pl.ds(0, BC)],
                          osem).wait()

  pltpu.async_copy(table_hbm.at[d], row_v, rsem)
  fire_idx(0, 0)
  pltpu.make_async_copy(table_hbm.at[d], row_v, rsem).wait()

  @pl.loop(0, NCHUNK)
  def _chunk(g):
    s = g % 2

    @pl.when(g + 1 < NCHUNK)
    def _():
      fire_idx(g + 1, 1 - s)

    wait_idx(s)

    @pl.when(g >= 2)
    def _():
      wait_out()

    idx_s = idx_v.at[s]
    out_s = out_v.at[s]

    @plsc.parallel_loop(0, HIST, unroll=5)
    def _h(h):
      for bs in range(BC // 16):
        v = idx_s[h, pl.ds(bs * 16, 16)]
        vals = plsc.load_gather(row_v, [v])
        out_s[h, pl.ds(bs * 16, 16)] = vals

    pltpu.async_copy(out_v.at[s], out_hbm.at[:, d, pl.ds(g * BC, BC)], osem)

  wait_out()
  wait_out()


def kernel(skills, embedding_table):
  idx_p = skills.T
  table_t = jnp.pad(embedding_table.T,
                    ((0, 0), (0, VPAD - embedding_table.shape[0])))
  out = _gather_t(idx_p, table_t)
  return jnp.transpose(out, (2, 0, 1))


# speedup vs baseline: 3.8288x; 1.0068x over previous
import functools

import jax
import jax.numpy as jnp
from jax import lax
from jax.experimental import pallas as pl
from jax.experimental.pallas import tpu as pltpu
from jax.experimental.pallas import tpu_sc as plsc

EMBED_DIM = 32
NUM_CORES = 2
NUM_SUBCORES = 16
NUM_WORKERS = NUM_CORES * NUM_SUBCORES
BATCH = 16384
HIST = 50
HIST_PAD = 128
VPAD = 100096
BC = 128
NCHUNK = BATCH // BC

_MESH = plsc.VectorSubcoreMesh(
    core_axis_name="c", subcore_axis_name="s",
    num_cores=NUM_CORES, num_subcores=NUM_SUBCORES,
)


@functools.partial(
    pl.kernel,
    mesh=_MESH,
    compiler_params=pltpu.CompilerParams(use_tc_tiling_on_sc=False, needs_layout_passes=False),
    out_type=jax.ShapeDtypeStruct((HIST, EMBED_DIM, BATCH), jnp.float32),
    scratch_types=[
        pltpu.VMEM((VPAD,), jnp.float32),
        pltpu.VMEM((2, HIST, BC), jnp.int32),
        pltpu.VMEM((2, HIST, BC), jnp.float32),
        pltpu.SemaphoreType.DMA,
        pltpu.SemaphoreType.DMA,
        pltpu.SemaphoreType.DMA,
    ],
)
def _gather_t(idx_hbm, table_hbm, out_hbm, row_v, idx_v, out_v,
              rsem, isem, osem):
  d = lax.axis_index("s") * NUM_CORES + lax.axis_index("c")

  def fire_idx(g, slot):
    pltpu.async_copy(idx_hbm.at[:, pl.ds(g * BC, BC)], idx_v.at[slot], isem)

  def wait_idx(slot):
    pltpu.make_async_copy(idx_hbm.at[:, pl.ds(0, BC)], idx_v.at[slot],
                          isem).wait()

  def wait_out():
    pltpu.make_async_copy(out_v.at[0], out_hbm.at[:, 0, pl.ds(0, BC)],
                          osem).wait()

  pltpu.async_copy(table_hbm.at[d], row_v, rsem)
  fire_idx(0, 0)
  pltpu.make_async_copy(table_hbm.at[d], row_v, rsem).wait()

  @pl.loop(0, NCHUNK)
  def _chunk(g):
    s = g % 2

    @pl.when(g + 1 < NCHUNK)
    def _():
      fire_idx(g + 1, 1 - s)

    wait_idx(s)

    @pl.when(g >= 2)
    def _():
      wait_out()

    idx_s = idx_v.at[s]
    out_s = out_v.at[s]

    @plsc.parallel_loop(0, HIST * (BC // 16), unroll=8)
    def _k(k):
      h = k // (BC // 16)
      bs = k % (BC // 16)
      v = idx_s[h, pl.ds(bs * 16, 16)]
      vals = plsc.load_gather(row_v, [v])
      out_s[h, pl.ds(bs * 16, 16)] = vals

    pltpu.async_copy(out_v.at[s], out_hbm.at[:, d, pl.ds(g * BC, BC)], osem)

  wait_out()
  wait_out()


def kernel(skills, embedding_table):
  idx_p = skills.T
  table_t = jnp.pad(embedding_table.T,
                    ((0, 0), (0, VPAD - embedding_table.shape[0])))
  out = _gather_t(idx_p, table_t)
  return jnp.transpose(out, (2, 0, 1))
